# Initial kernel scaffold; baseline (speedup 1.0000x reference)
import jax
import jax.numpy as jnp
from jax.experimental import pallas as pl


def kernel(A_AP, A_UE, H, Graph_AP_reshape, GFA_AP, Graph_UE_reshape, GFA_UE, Q1_AP, Q2_AP, Q1_UE, Q2_UE, P1_AP, P1_UE, Att_AP, Att_UE, bn_gamma, bn_beta, permutation_size1, permutation_size2, BATCH_SIZE):
    raise NotImplementedError("write your pallas kernel here")


# speedup vs baseline: 13.2456x; 13.2456x over previous
import numpy as np
import jax
import jax.numpy as jnp
from jax.experimental import pallas as pl
from jax.experimental.pallas import tpu as pltpu

_IN1 = 64
_IN2 = 64
_OUT = 64
_SAMPLE = 2
_LOG2PI = float(np.log(2.0 * np.pi))


def _phase1_body(aap_ref, aue_ref, h_ref, gap_ref, gfaap_ref, gue_ref, gfaue_ref,
                 q1ap_ref, q2ap_ref, q1ue_ref, q2ue_ref, p1ap_ref, p1ue_ref,
                 attap_ref, attue_ref, uap_ref, uue_ref,
                 preap_ref, preue_ref, alap_ref, alue_ref):
    f32 = jnp.float32
    n1 = aap_ref.shape[2]
    n2 = aue_ref.shape[2]

    def side(a, idx_i, idx_j, gfa, att, u, n_nodes, deg, n_edges):
        iota_n = jax.lax.broadcasted_iota(jnp.int32, (n_nodes, n_edges), 0)
        oh_i = (iota_n == idx_i).astype(f32)
        oh_j = (iota_n == idx_j).astype(f32)
        xiT = jnp.dot(a, oh_i, preferred_element_type=f32)
        xjT = jnp.dot(a, oh_j, preferred_element_type=f32)
        s = (jnp.dot(att[:, :_IN1], xiT, preferred_element_type=f32)
             + jnp.dot(att[:, _IN1:], xjT, preferred_element_type=f32))
        lrelu = jnp.where(s >= 0, s, 0.2 * s)
        alpha = jnp.clip(jax.nn.sigmoid(lrelu), 0.01, 0.99)
        logits = jnp.log(alpha) - jnp.log1p(-alpha)
        noise = jnp.log(u) - jnp.log1p(-u)
        bern = jax.nn.sigmoid((logits + noise) / 0.1)
        xg = xjT * bern
        iota_e = jax.lax.broadcasted_iota(jnp.int32, (n_nodes, n_edges), 1)
        cmat = jnp.zeros((n_nodes, n_edges), f32)
        for k in range(deg):
            cmat = cmat + (gfa[:, k:k + 1] == iota_e).astype(f32)
        agg = jax.lax.dot_general(xg, cmat, (((1,), (1,)), ((), ())),
                                  preferred_element_type=f32)
        return agg, alpha

    a_ap = aap_ref[0]
    a_ue = aue_ref[0]
    e_ap = gap_ref.shape[2]
    e_ue = gue_ref.shape[2]
    agg_ap, alpha_ap = side(a_ap, gap_ref[0, 1:2, :], gap_ref[0, 0:1, :],
                            gfaap_ref[0], attap_ref[...], uap_ref[0],
                            n1, gfaap_ref.shape[2], e_ap)
    agg_ue, alpha_ue = side(a_ue, gue_ref[0, 1:2, :], gue_ref[0, 0:1, :],
                            gfaue_ref[0], attue_ref[...], uue_ref[0],
                            n2, gfaue_ref.shape[2], e_ue)

    h = h_ref[0]
    hm1 = jnp.sum(h, axis=2) * (1.0 / n2)
    hm2 = jnp.sum(h, axis=1) * (1.0 / n1)

    mean_ap = jnp.sum(agg_ap, axis=1, keepdims=True) * (1.0 / n1)
    mean_ue = jnp.sum(agg_ue, axis=1, keepdims=True) * (1.0 / n2)

    f = jnp.float32
    a1 = jnp.dot(q1ap_ref[...], agg_ap, preferred_element_type=f)
    a2 = jnp.dot(q2ap_ref[...], mean_ue, preferred_element_type=f)
    a3 = jnp.dot(p1ap_ref[...], hm1, preferred_element_type=f)
    preap_ref[0] = jnp.maximum(2.0 * a1 + 2.0 * a2 + 0.1 * a3, 0.0)

    u1 = jnp.dot(q1ue_ref[...], agg_ue, preferred_element_type=f)
    u2 = jnp.dot(q2ue_ref[...], mean_ap, preferred_element_type=f)
    u3 = jnp.dot(p1ue_ref[...], hm2, preferred_element_type=f)
    preue_ref[0] = jnp.maximum(2.0 * u1 + 2.0 * u2 + 0.1 * u3, 0.0)

    alap_ref[0] = alpha_ap
    alue_ref[0] = alpha_ue


def _phase2_body(preap_ref, preue_ref, alap_ref, alue_ref,
                 epsap_ref, epsue_ref, g_ref, b_ref,
                 oap_ref, oue_ref, ixzap_ref, ixzue_ref, iazap_ref, iazue_ref):
    gamma = g_ref[:, 0:1]
    beta = b_ref[:, 0:1]

    def bn(x):
        bsz = x.shape[0] * x.shape[2]
        s = jnp.sum(jnp.sum(x, axis=2, keepdims=True), axis=0, keepdims=True)
        m = s * (1.0 / bsz)
        d = x - m
        v = jnp.sum(jnp.sum(d * d, axis=2, keepdims=True), axis=0,
                    keepdims=True) * (1.0 / bsz)
        return gamma[None] * d / jnp.sqrt(v + 1e-5) + beta[None]

    def ib(y, eps_ref):
        mean = y[:, :_OUT, :]
        std = jax.nn.softplus(y[:, _OUT:, :]) + 1e-10
        logstd = jnp.log(std)
        acc = None
        for si in range(_SAMPLE):
            z = mean + std * eps_ref[si]
            e1 = -((z - mean) ** 2) / (2.0 * std * std) - logstd
            diff = jnp.sum(e1 + 0.5 * z * z, axis=1)
            acc = diff if acc is None else acc + diff
        t = acc * (1.0 / _SAMPLE)
        return jnp.sum(t, axis=1, keepdims=True)

    def kl(al_ref):
        al = al_ref[:, 0, :]
        term = (al * jnp.log(al / 0.5)
                + (1.0 - al) * jnp.log((1.0 - al) / 0.5))
        return jnp.sum(term, axis=1, keepdims=True)

    bsz = preap_ref.shape[0]
    w = ixzap_ref.shape[1]
    y_ap = bn(preap_ref[...])
    y_ue = bn(preue_ref[...])
    oap_ref[...] = y_ap
    oue_ref[...] = y_ue
    ixzap_ref[...] = jnp.broadcast_to(ib(y_ap, epsap_ref), (bsz, w))
    ixzue_ref[...] = jnp.broadcast_to(ib(y_ue, epsue_ref), (bsz, w))
    iazap_ref[...] = jnp.broadcast_to(kl(alap_ref), (bsz, w))
    iazue_ref[...] = jnp.broadcast_to(kl(alue_ref), (bsz, w))


def kernel(A_AP, A_UE, H, Graph_AP_reshape, GFA_AP, Graph_UE_reshape, GFA_UE,
           Q1_AP, Q2_AP, Q1_UE, Q2_UE, P1_AP, P1_UE, Att_AP, Att_UE,
           bn_gamma, bn_beta, permutation_size1, permutation_size2, BATCH_SIZE):
    f32 = jnp.float32
    B, D, N1 = A_AP.shape
    N2 = A_UE.shape[2]
    E_ap = Graph_AP_reshape.shape[2]
    E_ue = Graph_UE_reshape.shape[2]
    deg_ap = GFA_AP.shape[2]
    deg_ue = GFA_UE.shape[2]
    O2 = Q1_AP.shape[0]

    kr = jax.random.key(42)
    k1, k2, k3, k4 = jax.random.split(kr, 4)
    u_ap = jax.random.uniform(k1, (B, E_ap), minval=1e-6, maxval=1.0 - 1e-6)
    u_ue = jax.random.uniform(k2, (B, E_ue), minval=1e-6, maxval=1.0 - 1e-6)
    eps_ap = jax.random.normal(k3, (_SAMPLE, B * N1, _OUT))
    eps_ue = jax.random.normal(k4, (_SAMPLE, B * N2, _OUT))
    eps_ap_t = eps_ap.reshape(_SAMPLE, B, N1, _OUT).transpose(0, 1, 3, 2)
    eps_ue_t = eps_ue.reshape(_SAMPLE, B, N2, _OUT).transpose(0, 1, 3, 2)

    gap = Graph_AP_reshape.astype(jnp.int32)
    gue = Graph_UE_reshape.astype(jnp.int32)
    gfaap = GFA_AP.astype(jnp.int32)
    gfaue = GFA_UE.astype(jnp.int32)
    att_ap = Att_AP.reshape(1, 2 * D).astype(f32)
    att_ue = Att_UE.reshape(1, 2 * D).astype(f32)
    u_ap3 = u_ap.reshape(B, 1, E_ap)
    u_ue3 = u_ue.reshape(B, 1, E_ue)
    gcol = jnp.broadcast_to(bn_gamma.reshape(O2, 1), (O2, 128)).astype(f32)
    bcol = jnp.broadcast_to(bn_beta.reshape(O2, 1), (O2, 128)).astype(f32)

    wspec = lambda shp: pl.BlockSpec(shp, lambda b: (0,) * len(shp))
    bspec = lambda shp: pl.BlockSpec(shp, lambda b: (b,) + (0,) * (len(shp) - 1))

    pre_ap, pre_ue, alpha_ap, alpha_ue = pl.pallas_call(
        _phase1_body,
        grid=(B,),
        in_specs=[
            bspec((1, D, N1)),
            bspec((1, D, N2)),
            bspec((1, _IN2, N1, N2)),
            bspec((1, 2, E_ap)),
            bspec((1, N1, deg_ap)),
            bspec((1, 2, E_ue)),
            bspec((1, N2, deg_ue)),
            wspec((O2, D)),
            wspec((O2, D)),
            wspec((O2, D)),
            wspec((O2, D)),
            wspec((O2, _IN2)),
            wspec((O2, _IN2)),
            wspec((1, 2 * D)),
            wspec((1, 2 * D)),
            bspec((1, 1, E_ap)),
            bspec((1, 1, E_ue)),
        ],
        out_specs=[
            bspec((1, O2, N1)),
            bspec((1, O2, N2)),
            bspec((1, 1, E_ap)),
            bspec((1, 1, E_ue)),
        ],
        out_shape=[
            jax.ShapeDtypeStruct((B, O2, N1), f32),
            jax.ShapeDtypeStruct((B, O2, N2), f32),
            jax.ShapeDtypeStruct((B, 1, E_ap), f32),
            jax.ShapeDtypeStruct((B, 1, E_ue), f32),
        ],
        compiler_params=pltpu.CompilerParams(
            dimension_semantics=("arbitrary",)),
    )(A_AP, A_UE, H, gap, gfaap, gue, gfaue,
      Q1_AP, Q2_AP, Q1_UE, Q2_UE, P1_AP, P1_UE, att_ap, att_ue, u_ap3, u_ue3)

    out_ap, out_ue, ixz_ap, ixz_ue, iaz_ap, iaz_ue = pl.pallas_call(
        _phase2_body,
        out_shape=[
            jax.ShapeDtypeStruct((B, O2, N1), f32),
            jax.ShapeDtypeStruct((B, O2, N2), f32),
            jax.ShapeDtypeStruct((B, 128), f32),
            jax.ShapeDtypeStruct((B, 128), f32),
            jax.ShapeDtypeStruct((B, 128), f32),
            jax.ShapeDtypeStruct((B, 128), f32),
        ],
    )(pre_ap, pre_ue, alpha_ap, alpha_ue, eps_ap_t, eps_ue_t, gcol, bcol)

    return (out_ap, out_ue, ixz_ap[:, 0], ixz_ue[:, 0],
            iaz_ap[:, 0], iaz_ue[:, 0])
